# baseline (speedup 1.0000x reference)
import functools

import jax
import jax.numpy as jnp
from jax import lax
from jax.experimental import pallas as pl
from jax.experimental.pallas import tpu as pltpu
from jax.experimental.pallas import tpu_sc as plsc

N = 10000
E = 320000
NF = 6

NC = 2
NS = 16
NW = NC * NS
E_PER_W = E // NW
CH = 80
E_PER_W_PAD = 10240
NCH = E_PER_W_PAD // CH
NBUF = 2
N_PAD = 10240
RPT = N_PAD // NS


def _mesh():
    return plsc.VectorSubcoreMesh(core_axis_name="c", subcore_axis_name="s")


def _deg_body(dst_hbm, ones_hbm, zeros_hbm, out_hbm, idx_v, ones_v, acc_sh):
    cid = lax.axis_index("c")
    sid = lax.axis_index("s")
    wid = sid * NC + cid
    r0 = sid * RPT
    pltpu.sync_copy(zeros_hbm.at[pl.ds(r0, RPT)], acc_sh.at[pl.ds(r0, RPT)])
    pltpu.sync_copy(ones_hbm, ones_v)
    pltpu.sync_copy(dst_hbm.at[wid], idx_v)
    plsc.subcore_barrier()

    def body(j, carry):
        pltpu.sync_copy(ones_v, acc_sh.at[idx_v.at[j]], add=True)
        return carry

    lax.fori_loop(0, NCH, body, 0)
    plsc.subcore_barrier()
    pltpu.sync_copy(acc_sh.at[pl.ds(r0, RPT)], out_hbm.at[cid, pl.ds(r0, RPT)])


@functools.lru_cache(maxsize=None)
def _deg_call_fn():
    return functools.partial(
        pl.kernel,
        out_type=jax.ShapeDtypeStruct((NC, N_PAD, 16), jnp.float32),
        mesh=_mesh(),
        scratch_types=[
            pltpu.VMEM((NCH, CH), jnp.int32),
            pltpu.VMEM((CH, 16), jnp.float32),
            pltpu.VMEM_SHARED((N_PAD, 16), jnp.float32),
        ],
        compiler_params=pltpu.CompilerParams(use_tc_tiling_on_sc=False),
    )(_deg_body)


def _agg_body(src_hbm, dst_hbm, g_hbm, zeros_hbm, out_hbm,
              sidx_v, didx_v, rows_v, acc_sh,
              sg0, sg1):
    cid = lax.axis_index("c")
    sid = lax.axis_index("s")
    wid = sid * NC + cid
    r0 = sid * RPT
    sg = (sg0, sg1)
    pltpu.sync_copy(zeros_hbm.at[pl.ds(r0, RPT)], acc_sh.at[pl.ds(r0, RPT)])
    pltpu.sync_copy(src_hbm.at[wid], sidx_v)
    pltpu.sync_copy(dst_hbm.at[wid], didx_v)
    plsc.subcore_barrier()

    def start_g(c, b):
        pltpu.async_copy(g_hbm.at[sidx_v.at[c]], rows_v.at[b], sg[b])

    def wait_g(b):
        pltpu.make_async_copy(g_hbm.at[pl.ds(0, CH)], rows_v.at[b], sg[b]).wait()

    def scat(c, b):
        pltpu.sync_copy(rows_v.at[b], acc_sh.at[didx_v.at[c]], add=True)

    for b in range(NBUF):
        start_g(b, b)

    def body(i, carry):
        c0 = NBUF * i
        for b in range(NBUF):
            wait_g(b)
            scat(c0 + b, b)
            start_g(c0 + NBUF + b, b)
        return carry

    lax.fori_loop(0, NCH // NBUF - 1, body, 0)
    for b in range(NBUF):
        wait_g(b)
        scat(NCH - NBUF + b, b)
    plsc.subcore_barrier()
    pltpu.sync_copy(acc_sh.at[pl.ds(r0, RPT)], out_hbm.at[cid, pl.ds(r0, RPT)])


@functools.lru_cache(maxsize=None)
def _agg_call_fn():
    return functools.partial(
        pl.kernel,
        out_type=jax.ShapeDtypeStruct((NC, N_PAD, 32), jnp.float32),
        mesh=_mesh(),
        scratch_types=[
            pltpu.VMEM((NCH, CH), jnp.int32),
            pltpu.VMEM((NCH, CH), jnp.int32),
            pltpu.VMEM((NBUF, CH, 32), jnp.float32),
            pltpu.VMEM_SHARED((N_PAD, 32), jnp.float32),
        ] + [pltpu.SemaphoreType.DMA] * NBUF,
        compiler_params=pltpu.CompilerParams(use_tc_tiling_on_sc=False),
    )(_agg_body)


BR = 80
K_FULL = 20000


def _tc1_body(x_ref, s_ref, r_ref, a1_ref, a2_ref, be1_ref, w2_ref, be2_ref,
              wg1_ref, out_ref):
    sa = jnp.dot(s_ref[...].astype(jnp.bfloat16),
                 r_ref[...].astype(jnp.bfloat16),
                 preferred_element_type=jnp.float32)
    h = jnp.tanh(
        jnp.dot(x_ref[...], a1_ref[...], preferred_element_type=jnp.float32)
        + jnp.dot(sa, a2_ref[...], preferred_element_type=jnp.float32)
        + be1_ref[...])
    h = jnp.tanh(
        jnp.dot(h, w2_ref[...], preferred_element_type=jnp.float32)
        + be2_ref[...])
    out_ref[...] = jnp.dot(h, wg1_ref[...], preferred_element_type=jnp.float32)


def _tc1(x, S, R, A1, A2, be1, W2, be2, Wg1):
    return pl.pallas_call(
        _tc1_body,
        grid=(N // BR,),
        in_specs=[
            pl.BlockSpec((BR, NF), lambda i: (i, 0)),
            pl.BlockSpec((BR, K_FULL), lambda i: (i, 0)),
            pl.BlockSpec((K_FULL, 3), lambda i: (0, 0)),
            pl.BlockSpec((NF, 64), lambda i: (0, 0)),
            pl.BlockSpec((3, 64), lambda i: (0, 0)),
            pl.BlockSpec((1, 64), lambda i: (0, 0)),
            pl.BlockSpec((64, 32), lambda i: (0, 0)),
            pl.BlockSpec((1, 32), lambda i: (0, 0)),
            pl.BlockSpec((32, 32), lambda i: (0, 0)),
        ],
        out_specs=pl.BlockSpec((BR, 32), lambda i: (i, 0)),
        out_shape=jax.ShapeDtypeStruct((N, 32), jnp.float32),
        compiler_params=pltpu.CompilerParams(
            dimension_semantics=("arbitrary",)),
    )(x, S, R, A1, A2, be1, W2, be2, Wg1)


def _prep_body(d0_ref, d1_ref, hw1_ref, g1_ref, dinv_ref):
    deg = d0_ref[:, 0:1] + d1_ref[:, 0:1] + 1.0
    dinv = lax.rsqrt(deg)
    dinv_ref[...] = dinv
    g1_ref[...] = hw1_ref[...] * dinv


def _prep(d0, d1, hw1):
    return pl.pallas_call(
        _prep_body,
        out_shape=(jax.ShapeDtypeStruct((N, 32), jnp.float32),
                   jax.ShapeDtypeStruct((N, 1), jnp.float32)),
    )(d0, d1, hw1)


def _mid_body(a0_ref, a1_ref, hw_ref, dinv_ref, b_ref, w_ref, hwn_ref, gn_ref):
    dinv = dinv_ref[...]
    h = jnp.maximum(
        dinv * (a0_ref[...] + a1_ref[...]) + dinv * dinv * hw_ref[...] + b_ref[...],
        0.0)
    hwn = jnp.dot(h, w_ref[...], preferred_element_type=jnp.float32)
    hwn_ref[...] = hwn
    gn_ref[...] = hwn * dinv


def _mid(a0, a1, hw, dinv, b, w):
    return pl.pallas_call(
        _mid_body,
        out_shape=(jax.ShapeDtypeStruct((N, 32), jnp.float32),
                   jax.ShapeDtypeStruct((N, 32), jnp.float32)),
    )(a0, a1, hw, dinv, b, w)


def _fin_body(a0_ref, a1_ref, hw_ref, dinv_ref, b_ref, wp1_ref, bp1_ref,
              wp2_ref, bp2_ref, out_ref):
    dinv = dinv_ref[...]
    h = jnp.maximum(
        dinv * (a0_ref[...] + a1_ref[...]) + dinv * dinv * hw_ref[...] + b_ref[...],
        0.0)
    t = jnp.tanh(jnp.dot(h, wp1_ref[...], preferred_element_type=jnp.float32)
                 + bp1_ref[...])
    out_ref[...] = jnp.tanh(
        jnp.dot(t, wp2_ref[...], preferred_element_type=jnp.float32)
        + bp2_ref[...])


def _fin(a0, a1, hw, dinv, b, Wp1, bp1, Wp2, bp2):
    return pl.pallas_call(
        _fin_body,
        out_shape=jax.ShapeDtypeStruct((N, 1), jnp.float32),
    )(a0, a1, hw, dinv, b, Wp1, bp1, Wp2, bp2)


def kernel(x, edge_index, S, R, We1, be1, We2, be2, Wg1, bg1, Wg2, bg2,
           Wp1, bp1, Wp2, bp2):
    src2 = edge_index[0].reshape(NW, E_PER_W)
    dst2 = edge_index[1].reshape(NW, E_PER_W)
    pad = E_PER_W_PAD - E_PER_W
    src3 = jnp.pad(src2, ((0, 0), (0, pad))).reshape(NW, NCH, CH)
    dst3 = jnp.pad(dst2, ((0, 0), (0, pad)),
                   constant_values=N).reshape(NW, NCH, CH)
    zeros16 = jnp.zeros((N_PAD, 16), jnp.float32)
    zeros32 = jnp.zeros((N_PAD, 32), jnp.float32)
    ones16 = jnp.ones((CH, 16), jnp.float32)

    degacc = _deg_call_fn()(dst3, ones16, zeros16)

    A1 = We1[:NF]
    A2 = We1[NF:]
    hw1 = _tc1(x, S, R, A1, A2, be1.reshape(1, 64), We2,
               be2.reshape(1, 32), Wg1)

    g1, dinv = _prep(degacc[0, :N], degacc[1, :N], hw1)

    acc1 = _agg_call_fn()(src3, dst3, g1, zeros32)
    hw2, g2 = _mid(acc1[0, :N], acc1[1, :N], hw1, dinv, bg1.reshape(1, 32), Wg2)

    acc2 = _agg_call_fn()(src3, dst3, g2, zeros32)
    return _fin(acc2[0, :N], acc2[1, :N], hw2, dinv, bg2.reshape(1, 32),
                Wp1, bp1.reshape(1, 32), Wp2, bp2.reshape(1, 1))


# speedup vs baseline: 1.0003x; 1.0003x over previous
import functools

import jax
import jax.numpy as jnp
from jax import lax
from jax.experimental import pallas as pl
from jax.experimental.pallas import tpu as pltpu
from jax.experimental.pallas import tpu_sc as plsc

N = 10000
E = 320000
NF = 6

NC = 2
NS = 16
NW = NC * NS
E_PER_W = E // NW
CH = 80
E_PER_W_PAD = 10240
NCH = E_PER_W_PAD // CH
NBUF = 2
N_PAD = 10240
RPT = N_PAD // NS


def _mesh():
    return plsc.VectorSubcoreMesh(core_axis_name="c", subcore_axis_name="s")


def _deg_body(dst_hbm, ones_hbm, zeros_hbm, out_hbm, idx_v, ones_v, acc_sh):
    cid = lax.axis_index("c")
    sid = lax.axis_index("s")
    wid = sid * NC + cid
    r0 = sid * RPT
    pltpu.sync_copy(zeros_hbm.at[pl.ds(r0, RPT)], acc_sh.at[pl.ds(r0, RPT)])
    pltpu.sync_copy(ones_hbm, ones_v)
    pltpu.sync_copy(dst_hbm.at[wid], idx_v)
    plsc.subcore_barrier()

    def body(j, carry):
        pltpu.sync_copy(ones_v, acc_sh.at[idx_v.at[j]], add=True)
        return carry

    lax.fori_loop(0, NCH, body, 0)
    plsc.subcore_barrier()
    pltpu.sync_copy(acc_sh.at[pl.ds(r0, RPT)], out_hbm.at[cid, pl.ds(r0, RPT)])


@functools.lru_cache(maxsize=None)
def _deg_call_fn():
    return functools.partial(
        pl.kernel,
        out_type=jax.ShapeDtypeStruct((NC, N_PAD, 16), jnp.float32),
        mesh=_mesh(),
        scratch_types=[
            pltpu.VMEM((NCH, CH), jnp.int32),
            pltpu.VMEM((CH, 16), jnp.float32),
            pltpu.VMEM_SHARED((N_PAD, 16), jnp.float32),
        ],
        compiler_params=pltpu.CompilerParams(use_tc_tiling_on_sc=False),
    )(_deg_body)


def _agg_body(src_hbm, dst_hbm, g_hbm, zeros_hbm, out_hbm,
              sidx_v, didx_v, rows_v, acc_sh,
              sg0, sg1):
    cid = lax.axis_index("c")
    sid = lax.axis_index("s")
    wid = sid * NC + cid
    r0 = sid * RPT
    sg = (sg0, sg1)
    pltpu.sync_copy(zeros_hbm.at[pl.ds(r0, RPT)], acc_sh.at[pl.ds(r0, RPT)])
    pltpu.sync_copy(src_hbm.at[wid], sidx_v)
    pltpu.sync_copy(dst_hbm.at[wid], didx_v)
    plsc.subcore_barrier()

    def start_g(c, b):
        pltpu.async_copy(g_hbm.at[sidx_v.at[c]], rows_v.at[b], sg[b])

    def wait_g(b):
        pltpu.make_async_copy(g_hbm.at[pl.ds(0, CH)], rows_v.at[b], sg[b]).wait()

    def scat(c, b):
        pltpu.sync_copy(rows_v.at[b], acc_sh.at[didx_v.at[c]], add=True)

    for b in range(NBUF):
        start_g(b, b)

    def body(i, carry):
        c0 = NBUF * i
        for b in range(NBUF):
            wait_g(b)
            scat(c0 + b, b)
            start_g(c0 + NBUF + b, b)
        return carry

    lax.fori_loop(0, NCH // NBUF - 1, body, 0)
    for b in range(NBUF):
        wait_g(b)
        scat(NCH - NBUF + b, b)
    plsc.subcore_barrier()
    pltpu.sync_copy(acc_sh.at[pl.ds(r0, RPT)], out_hbm.at[cid, pl.ds(r0, RPT)])


@functools.lru_cache(maxsize=None)
def _agg_call_fn():
    return functools.partial(
        pl.kernel,
        out_type=jax.ShapeDtypeStruct((NC, N_PAD, 32), jnp.float32),
        mesh=_mesh(),
        scratch_types=[
            pltpu.VMEM((NCH, CH), jnp.int32),
            pltpu.VMEM((NCH, CH), jnp.int32),
            pltpu.VMEM((NBUF, CH, 32), jnp.float32),
            pltpu.VMEM_SHARED((N_PAD, 32), jnp.float32),
        ] + [pltpu.SemaphoreType.DMA] * NBUF,
        compiler_params=pltpu.CompilerParams(use_tc_tiling_on_sc=False),
    )(_agg_body)


BR = 80
K_FULL = 20000


def _tc1_body(x_ref, s_ref, r_ref, a1_ref, a2_ref, be1_ref, w2_ref, be2_ref,
              wg1_ref, out_ref):
    sa = jnp.dot(s_ref[...].astype(jnp.bfloat16),
                 r_ref[...].astype(jnp.bfloat16),
                 preferred_element_type=jnp.float32)
    h = jnp.tanh(
        jnp.dot(x_ref[...], a1_ref[...], preferred_element_type=jnp.float32)
        + jnp.dot(sa, a2_ref[...], preferred_element_type=jnp.float32)
        + be1_ref[...])
    h = jnp.tanh(
        jnp.dot(h, w2_ref[...], preferred_element_type=jnp.float32)
        + be2_ref[...])
    out_ref[...] = jnp.dot(h, wg1_ref[...], preferred_element_type=jnp.float32)


def _tc1(x, S, R, A1, A2, be1, W2, be2, Wg1):
    return pl.pallas_call(
        _tc1_body,
        grid=(N // BR,),
        in_specs=[
            pl.BlockSpec((BR, NF), lambda i: (i, 0)),
            pl.BlockSpec((BR, K_FULL), lambda i: (i, 0)),
            pl.BlockSpec((K_FULL, 3), lambda i: (0, 0)),
            pl.BlockSpec((NF, 64), lambda i: (0, 0)),
            pl.BlockSpec((3, 64), lambda i: (0, 0)),
            pl.BlockSpec((1, 64), lambda i: (0, 0)),
            pl.BlockSpec((64, 32), lambda i: (0, 0)),
            pl.BlockSpec((1, 32), lambda i: (0, 0)),
            pl.BlockSpec((32, 32), lambda i: (0, 0)),
        ],
        out_specs=pl.BlockSpec((BR, 32), lambda i: (i, 0)),
        out_shape=jax.ShapeDtypeStruct((N, 32), jnp.float32),
        compiler_params=pltpu.CompilerParams(
            dimension_semantics=("arbitrary",)),
    )(x, S, R, A1, A2, be1, W2, be2, Wg1)


def _prep_body(d0_ref, d1_ref, hw1_ref, g1_ref, dinv_ref):
    deg = d0_ref[:, 0:1] + d1_ref[:, 0:1] + 1.0
    dinv = lax.rsqrt(deg)
    dinv_ref[...] = dinv
    g1_ref[...] = hw1_ref[...] * dinv


def _prep(d0, d1, hw1):
    return pl.pallas_call(
        _prep_body,
        out_shape=(jax.ShapeDtypeStruct((N, 32), jnp.float32),
                   jax.ShapeDtypeStruct((N, 1), jnp.float32)),
    )(d0, d1, hw1)


def _mid_body(a0_ref, a1_ref, hw_ref, dinv_ref, b_ref, w_ref, hwn_ref, gn_ref):
    dinv = dinv_ref[...]
    h = jnp.maximum(
        dinv * (a0_ref[...] + a1_ref[...]) + dinv * dinv * hw_ref[...] + b_ref[...],
        0.0)
    hwn = jnp.dot(h, w_ref[...], preferred_element_type=jnp.float32)
    hwn_ref[...] = hwn
    gn_ref[...] = hwn * dinv


def _mid(a0, a1, hw, dinv, b, w):
    return pl.pallas_call(
        _mid_body,
        out_shape=(jax.ShapeDtypeStruct((N, 32), jnp.float32),
                   jax.ShapeDtypeStruct((N, 32), jnp.float32)),
    )(a0, a1, hw, dinv, b, w)


def _fin_body(a0_ref, a1_ref, hw_ref, dinv_ref, b_ref, wp1_ref, bp1_ref,
              wp2_ref, bp2_ref, out_ref):
    dinv = dinv_ref[...]
    h = jnp.maximum(
        dinv * (a0_ref[...] + a1_ref[...]) + dinv * dinv * hw_ref[...] + b_ref[...],
        0.0)
    t = jnp.tanh(jnp.dot(h, wp1_ref[...], preferred_element_type=jnp.float32)
                 + bp1_ref[...])
    out_ref[...] = jnp.tanh(
        jnp.dot(t, wp2_ref[...], preferred_element_type=jnp.float32)
        + bp2_ref[...])


def _fin(a0, a1, hw, dinv, b, Wp1, bp1, Wp2, bp2):
    return pl.pallas_call(
        _fin_body,
        out_shape=jax.ShapeDtypeStruct((N, 1), jnp.float32),
    )(a0, a1, hw, dinv, b, Wp1, bp1, Wp2, bp2)


def kernel(x, edge_index, S, R, We1, be1, We2, be2, Wg1, bg1, Wg2, bg2,
           Wp1, bp1, Wp2, bp2):
    src2 = edge_index[0].reshape(NW, E_PER_W)
    dst2 = edge_index[1].reshape(NW, E_PER_W)
    pad = E_PER_W_PAD - E_PER_W
    pad_dst = jnp.broadcast_to(N + jnp.arange(pad, dtype=jnp.int32),
                               (NW, pad))
    src3 = jnp.pad(src2, ((0, 0), (0, pad))).reshape(NW, NCH, CH)
    dst3 = jnp.concatenate([dst2, pad_dst], axis=1).reshape(NW, NCH, CH)
    zeros16 = jnp.zeros((N_PAD, 16), jnp.float32)
    zeros32 = jnp.zeros((N_PAD, 32), jnp.float32)
    ones16 = jnp.ones((CH, 16), jnp.float32)

    degacc = _deg_call_fn()(dst3, ones16, zeros16)

    A1 = We1[:NF]
    A2 = We1[NF:]
    hw1 = _tc1(x, S, R, A1, A2, be1.reshape(1, 64), We2,
               be2.reshape(1, 32), Wg1)

    g1, dinv = _prep(degacc[0, :N], degacc[1, :N], hw1)

    acc1 = _agg_call_fn()(src3, dst3, g1, zeros32)
    hw2, g2 = _mid(acc1[0, :N], acc1[1, :N], hw1, dinv, bg1.reshape(1, 32), Wg2)

    acc2 = _agg_call_fn()(src3, dst3, g2, zeros32)
    return _fin(acc2[0, :N], acc2[1, :N], hw2, dinv, bg2.reshape(1, 32),
                Wp1, bp1.reshape(1, 32), Wp2, bp2.reshape(1, 1))


# speedup vs baseline: 1.1615x; 1.1612x over previous
import functools

import jax
import jax.numpy as jnp
from jax import lax
from jax.experimental import pallas as pl
from jax.experimental.pallas import tpu as pltpu
from jax.experimental.pallas import tpu_sc as plsc

N = 10000
E = 320000
NF = 6

NC = 2
NS = 16
NW = NC * NS
E_PER_W = E // NW
CH = 80
E_PER_W_PAD = 10080
NCH = E_PER_W_PAD // CH
N_PAD = 10240
RPT = N_PAD // NS


def _mesh():
    return plsc.VectorSubcoreMesh(core_axis_name="c", subcore_axis_name="s")


def _deg_body(dst_hbm, ones_hbm, zeros_hbm, out_hbm, idx_v, ones_v, acc_sh):
    cid = lax.axis_index("c")
    sid = lax.axis_index("s")
    wid = sid * NC + cid
    r0 = sid * RPT
    pltpu.sync_copy(zeros_hbm.at[pl.ds(r0, RPT)], acc_sh.at[pl.ds(r0, RPT)])
    pltpu.sync_copy(ones_hbm, ones_v)
    pltpu.sync_copy(dst_hbm.at[wid], idx_v)
    plsc.subcore_barrier()

    def body(j, carry):
        pltpu.sync_copy(ones_v, acc_sh.at[idx_v.at[j]], add=True)
        return carry

    lax.fori_loop(0, NCH, body, 0)
    plsc.subcore_barrier()
    pltpu.sync_copy(acc_sh.at[pl.ds(r0, RPT)], out_hbm.at[cid, pl.ds(r0, RPT)])


@functools.lru_cache(maxsize=None)
def _deg_call_fn():
    return functools.partial(
        pl.kernel,
        out_type=jax.ShapeDtypeStruct((NC, N_PAD, 16), jnp.float32),
        mesh=_mesh(),
        scratch_types=[
            pltpu.VMEM((NCH, CH), jnp.int32),
            pltpu.VMEM((CH, 16), jnp.float32),
            pltpu.VMEM_SHARED((N_PAD, 16), jnp.float32),
        ],
        compiler_params=pltpu.CompilerParams(use_tc_tiling_on_sc=False),
    )(_deg_body)


def _agg_body(src_hbm, dst_hbm, g_hbm, zeros_hbm, out_hbm,
              sidx_v, didx_v, rows_v, acc_sh, sem0, sem1):
    cid = lax.axis_index("c")
    sid = lax.axis_index("s")
    wid = sid * NC + cid
    r0 = sid * RPT
    pltpu.sync_copy(zeros_hbm.at[pl.ds(r0, RPT)], acc_sh.at[pl.ds(r0, RPT)])
    pltpu.sync_copy(src_hbm.at[wid], sidx_v)
    pltpu.sync_copy(dst_hbm.at[wid], didx_v)
    plsc.subcore_barrier()

    def start_g(c, b, sem):
        pltpu.async_copy(g_hbm.at[sidx_v.at[c]], rows_v.at[b], sem)

    def wait_g(b, sem):
        pltpu.make_async_copy(g_hbm.at[pl.ds(0, CH)], rows_v.at[b], sem).wait()

    def scat(c, b):
        pltpu.sync_copy(rows_v.at[b], acc_sh.at[didx_v.at[c]], add=True)

    start_g(0, 0, sem0)
    start_g(1, 1, sem1)

    def body(i, carry):
        c0 = 2 * i
        wait_g(0, sem0)
        scat(c0, 0)
        start_g(c0 + 2, 0, sem0)
        wait_g(1, sem1)
        scat(c0 + 1, 1)
        start_g(c0 + 3, 1, sem1)
        return carry

    lax.fori_loop(0, (NCH - 2) // 2, body, 0)
    wait_g(0, sem0)
    scat(NCH - 2, 0)
    wait_g(1, sem1)
    scat(NCH - 1, 1)
    plsc.subcore_barrier()
    pltpu.sync_copy(acc_sh.at[pl.ds(r0, RPT)], out_hbm.at[cid, pl.ds(r0, RPT)])


@functools.lru_cache(maxsize=None)
def _agg_call_fn():
    return functools.partial(
        pl.kernel,
        out_type=jax.ShapeDtypeStruct((NC, N_PAD, 32), jnp.float32),
        mesh=_mesh(),
        scratch_types=[
            pltpu.VMEM((NCH, CH), jnp.int32),
            pltpu.VMEM((NCH, CH), jnp.int32),
            pltpu.VMEM((2, CH, 32), jnp.float32),
            pltpu.VMEM_SHARED((N_PAD, 32), jnp.float32),
            pltpu.SemaphoreType.DMA,
            pltpu.SemaphoreType.DMA,
        ],
        compiler_params=pltpu.CompilerParams(use_tc_tiling_on_sc=False),
    )(_agg_body)


BR = 80
K_FULL = 20000


def _tc1_body(x_ref, s_ref, r_ref, a1_ref, a2_ref, be1_ref, w2_ref, be2_ref,
              wg1_ref, out_ref):
    sa = jnp.dot(s_ref[...].astype(jnp.bfloat16),
                 r_ref[...].astype(jnp.bfloat16),
                 preferred_element_type=jnp.float32)
    h = jnp.tanh(
        jnp.dot(x_ref[...], a1_ref[...], preferred_element_type=jnp.float32)
        + jnp.dot(sa, a2_ref[...], preferred_element_type=jnp.float32)
        + be1_ref[...])
    h = jnp.tanh(
        jnp.dot(h, w2_ref[...], preferred_element_type=jnp.float32)
        + be2_ref[...])
    out_ref[...] = jnp.dot(h, wg1_ref[...], preferred_element_type=jnp.float32)


def _tc1(x, S, R, A1, A2, be1, W2, be2, Wg1):
    return pl.pallas_call(
        _tc1_body,
        grid=(N // BR,),
        in_specs=[
            pl.BlockSpec((BR, NF), lambda i: (i, 0)),
            pl.BlockSpec((BR, K_FULL), lambda i: (i, 0)),
            pl.BlockSpec((K_FULL, 3), lambda i: (0, 0)),
            pl.BlockSpec((NF, 64), lambda i: (0, 0)),
            pl.BlockSpec((3, 64), lambda i: (0, 0)),
            pl.BlockSpec((1, 64), lambda i: (0, 0)),
            pl.BlockSpec((64, 32), lambda i: (0, 0)),
            pl.BlockSpec((1, 32), lambda i: (0, 0)),
            pl.BlockSpec((32, 32), lambda i: (0, 0)),
        ],
        out_specs=pl.BlockSpec((BR, 32), lambda i: (i, 0)),
        out_shape=jax.ShapeDtypeStruct((N, 32), jnp.float32),
        compiler_params=pltpu.CompilerParams(
            dimension_semantics=("arbitrary",)),
    )(x, S, R, A1, A2, be1, W2, be2, Wg1)


def _prep_body(d0_ref, d1_ref, hw1_ref, g1_ref, dinv_ref):
    deg = d0_ref[:, 0:1] + d1_ref[:, 0:1] + 1.0
    dinv = lax.rsqrt(deg)
    dinv_ref[...] = dinv
    g1_ref[...] = hw1_ref[...] * dinv


def _prep(d0, d1, hw1):
    return pl.pallas_call(
        _prep_body,
        out_shape=(jax.ShapeDtypeStruct((N, 32), jnp.float32),
                   jax.ShapeDtypeStruct((N, 1), jnp.float32)),
    )(d0, d1, hw1)


def _mid_body(a0_ref, a1_ref, hw_ref, dinv_ref, b_ref, w_ref, hwn_ref, gn_ref):
    dinv = dinv_ref[...]
    h = jnp.maximum(
        dinv * (a0_ref[...] + a1_ref[...]) + dinv * dinv * hw_ref[...] + b_ref[...],
        0.0)
    hwn = jnp.dot(h, w_ref[...], preferred_element_type=jnp.float32)
    hwn_ref[...] = hwn
    gn_ref[...] = hwn * dinv


def _mid(a0, a1, hw, dinv, b, w):
    return pl.pallas_call(
        _mid_body,
        out_shape=(jax.ShapeDtypeStruct((N, 32), jnp.float32),
                   jax.ShapeDtypeStruct((N, 32), jnp.float32)),
    )(a0, a1, hw, dinv, b, w)


def _fin_body(a0_ref, a1_ref, hw_ref, dinv_ref, b_ref, wp1_ref, bp1_ref,
              wp2_ref, bp2_ref, out_ref):
    dinv = dinv_ref[...]
    h = jnp.maximum(
        dinv * (a0_ref[...] + a1_ref[...]) + dinv * dinv * hw_ref[...] + b_ref[...],
        0.0)
    t = jnp.tanh(jnp.dot(h, wp1_ref[...], preferred_element_type=jnp.float32)
                 + bp1_ref[...])
    out_ref[...] = jnp.tanh(
        jnp.dot(t, wp2_ref[...], preferred_element_type=jnp.float32)
        + bp2_ref[...])


def _fin(a0, a1, hw, dinv, b, Wp1, bp1, Wp2, bp2):
    return pl.pallas_call(
        _fin_body,
        out_shape=jax.ShapeDtypeStruct((N, 1), jnp.float32),
    )(a0, a1, hw, dinv, b, Wp1, bp1, Wp2, bp2)


def kernel(x, edge_index, S, R, We1, be1, We2, be2, Wg1, bg1, Wg2, bg2,
           Wp1, bp1, Wp2, bp2):
    src2 = edge_index[0].reshape(NW, E_PER_W)
    dst2 = edge_index[1].reshape(NW, E_PER_W)
    pad = E_PER_W_PAD - E_PER_W
    pad_dst = jnp.broadcast_to(N + jnp.arange(pad, dtype=jnp.int32),
                               (NW, pad))
    src3 = jnp.pad(src2, ((0, 0), (0, pad))).reshape(NW, NCH, CH)
    dst3 = jnp.concatenate([dst2, pad_dst], axis=1).reshape(NW, NCH, CH)
    zeros16 = jnp.zeros((N_PAD, 16), jnp.float32)
    zeros32 = jnp.zeros((N_PAD, 32), jnp.float32)
    ones16 = jnp.ones((CH, 16), jnp.float32)

    degacc = _deg_call_fn()(dst3, ones16, zeros16)

    A1 = We1[:NF]
    A2 = We1[NF:]
    hw1 = _tc1(x, S, R, A1, A2, be1.reshape(1, 64), We2,
               be2.reshape(1, 32), Wg1)

    g1, dinv = _prep(degacc[0, :N], degacc[1, :N], hw1)

    acc1 = _agg_call_fn()(src3, dst3, g1, zeros32)
    hw2, g2 = _mid(acc1[0, :N], acc1[1, :N], hw1, dinv, bg1.reshape(1, 32), Wg2)

    acc2 = _agg_call_fn()(src3, dst3, g2, zeros32)
    return _fin(acc2[0, :N], acc2[1, :N], hw2, dinv, bg2.reshape(1, 32),
                Wp1, bp1.reshape(1, 32), Wp2, bp2.reshape(1, 1))


# speedup vs baseline: 1.2359x; 1.0640x over previous
import functools

import jax
import jax.numpy as jnp
from jax import lax
from jax.experimental import pallas as pl
from jax.experimental.pallas import tpu as pltpu
from jax.experimental.pallas import tpu_sc as plsc

N = 10000
E = 320000
NF = 6

NC = 2
NS = 16
NW = NC * NS
E_PER_W = E // NW
CH = 80
E_PER_W_PAD = 10080
NCH = E_PER_W_PAD // CH
N_PAD = 10240
RPT = N_PAD // NS


def _mesh():
    return plsc.VectorSubcoreMesh(core_axis_name="c", subcore_axis_name="s")


def _deg_body(dst_hbm, ones_hbm, zeros_hbm, out_hbm, idx_v, ones_v, acc_sh):
    cid = lax.axis_index("c")
    sid = lax.axis_index("s")
    wid = sid * NC + cid
    r0 = sid * RPT
    pltpu.sync_copy(zeros_hbm.at[pl.ds(r0, RPT)], acc_sh.at[pl.ds(r0, RPT)])
    pltpu.sync_copy(ones_hbm, ones_v)
    pltpu.sync_copy(dst_hbm.at[wid], idx_v)
    plsc.subcore_barrier()

    def body(j, carry):
        pltpu.sync_copy(ones_v, acc_sh.at[idx_v.at[j]], add=True)
        return carry

    lax.fori_loop(0, NCH, body, 0)
    plsc.subcore_barrier()
    pltpu.sync_copy(acc_sh.at[pl.ds(r0, RPT)], out_hbm.at[cid, pl.ds(r0, RPT)])


@functools.lru_cache(maxsize=None)
def _deg_call_fn():
    return functools.partial(
        pl.kernel,
        out_type=jax.ShapeDtypeStruct((NC, N_PAD, 16), jnp.float32),
        mesh=_mesh(),
        scratch_types=[
            pltpu.VMEM((NCH, CH), jnp.int32),
            pltpu.VMEM((CH, 16), jnp.float32),
            pltpu.VMEM_SHARED((N_PAD, 16), jnp.float32),
        ],
        compiler_params=pltpu.CompilerParams(use_tc_tiling_on_sc=False),
    )(_deg_body)


def _agg_body(src_hbm, dst_hbm, g_hbm, zeros_hbm, out_hbm,
              sidx_v, didx_v, rows_v, acc_sh, sem0, sem1):
    cid = lax.axis_index("c")
    sid = lax.axis_index("s")
    wid = sid * NC + cid
    r0 = sid * RPT
    pltpu.sync_copy(zeros_hbm.at[pl.ds(r0, RPT)], acc_sh.at[pl.ds(r0, RPT)])
    pltpu.sync_copy(src_hbm.at[wid], sidx_v)
    pltpu.sync_copy(dst_hbm.at[wid], didx_v)
    plsc.subcore_barrier()

    def start_g(c, b, sem):
        pltpu.async_copy(g_hbm.at[sidx_v.at[c]], rows_v.at[b], sem)

    def wait_g(b, sem):
        pltpu.make_async_copy(g_hbm.at[pl.ds(0, CH)], rows_v.at[b], sem).wait()

    def scat(c, b):
        pltpu.sync_copy(rows_v.at[b], acc_sh.at[didx_v.at[c]], add=True)

    start_g(0, 0, sem0)
    start_g(1, 1, sem1)

    def body(i, carry):
        c0 = 2 * i
        wait_g(0, sem0)
        scat(c0, 0)
        start_g(c0 + 2, 0, sem0)
        wait_g(1, sem1)
        scat(c0 + 1, 1)
        start_g(c0 + 3, 1, sem1)
        return carry

    lax.fori_loop(0, (NCH - 2) // 2, body, 0)
    wait_g(0, sem0)
    scat(NCH - 2, 0)
    wait_g(1, sem1)
    scat(NCH - 1, 1)
    plsc.subcore_barrier()
    pltpu.sync_copy(acc_sh.at[pl.ds(r0, RPT)], out_hbm.at[cid, pl.ds(r0, RPT)])


@functools.lru_cache(maxsize=None)
def _agg_call_fn():
    return functools.partial(
        pl.kernel,
        out_type=jax.ShapeDtypeStruct((NC, N_PAD, 32), jnp.float32),
        mesh=_mesh(),
        scratch_types=[
            pltpu.VMEM((NCH, CH), jnp.int32),
            pltpu.VMEM((NCH, CH), jnp.int32),
            pltpu.VMEM((2, CH, 32), jnp.float32),
            pltpu.VMEM_SHARED((N_PAD, 32), jnp.float32),
            pltpu.SemaphoreType.DMA,
            pltpu.SemaphoreType.DMA,
        ],
        compiler_params=pltpu.CompilerParams(use_tc_tiling_on_sc=False),
    )(_agg_body)


BR = 200
K_FULL = 20000


def _tc1_body(x_ref, s_ref, r_ref, a1_ref, a2_ref, be1_ref, w2_ref, be2_ref,
              wg1_ref, out_ref):
    sa = jnp.dot(s_ref[...].astype(jnp.bfloat16),
                 r_ref[...].astype(jnp.bfloat16),
                 preferred_element_type=jnp.float32)
    h = jnp.tanh(
        jnp.dot(x_ref[...], a1_ref[...], preferred_element_type=jnp.float32)
        + jnp.dot(sa, a2_ref[...], preferred_element_type=jnp.float32)
        + be1_ref[...])
    h = jnp.tanh(
        jnp.dot(h, w2_ref[...], preferred_element_type=jnp.float32)
        + be2_ref[...])
    out_ref[...] = jnp.dot(h, wg1_ref[...], preferred_element_type=jnp.float32)


def _tc1(x, S, R, A1, A2, be1, W2, be2, Wg1):
    return pl.pallas_call(
        _tc1_body,
        grid=(N // BR,),
        in_specs=[
            pl.BlockSpec((BR, NF), lambda i: (i, 0)),
            pl.BlockSpec((BR, K_FULL), lambda i: (i, 0)),
            pl.BlockSpec((K_FULL, 3), lambda i: (0, 0)),
            pl.BlockSpec((NF, 64), lambda i: (0, 0)),
            pl.BlockSpec((3, 64), lambda i: (0, 0)),
            pl.BlockSpec((1, 64), lambda i: (0, 0)),
            pl.BlockSpec((64, 32), lambda i: (0, 0)),
            pl.BlockSpec((1, 32), lambda i: (0, 0)),
            pl.BlockSpec((32, 32), lambda i: (0, 0)),
        ],
        out_specs=pl.BlockSpec((BR, 32), lambda i: (i, 0)),
        out_shape=jax.ShapeDtypeStruct((N, 32), jnp.float32),
        compiler_params=pltpu.CompilerParams(
            dimension_semantics=("arbitrary",)),
    )(x, S, R, A1, A2, be1, W2, be2, Wg1)


def _prep_body(d0_ref, d1_ref, hw1_ref, g1_ref, dinv_ref):
    deg = d0_ref[:, 0:1] + d1_ref[:, 0:1] + 1.0
    dinv = lax.rsqrt(deg)
    dinv_ref[...] = dinv
    g1_ref[...] = hw1_ref[...] * dinv


def _prep(d0, d1, hw1):
    return pl.pallas_call(
        _prep_body,
        out_shape=(jax.ShapeDtypeStruct((N, 32), jnp.float32),
                   jax.ShapeDtypeStruct((N, 1), jnp.float32)),
    )(d0, d1, hw1)


def _mid_body(a0_ref, a1_ref, hw_ref, dinv_ref, b_ref, w_ref, hwn_ref, gn_ref):
    dinv = dinv_ref[...]
    h = jnp.maximum(
        dinv * (a0_ref[...] + a1_ref[...]) + dinv * dinv * hw_ref[...] + b_ref[...],
        0.0)
    hwn = jnp.dot(h, w_ref[...], preferred_element_type=jnp.float32)
    hwn_ref[...] = hwn
    gn_ref[...] = hwn * dinv


def _mid(a0, a1, hw, dinv, b, w):
    return pl.pallas_call(
        _mid_body,
        out_shape=(jax.ShapeDtypeStruct((N, 32), jnp.float32),
                   jax.ShapeDtypeStruct((N, 32), jnp.float32)),
    )(a0, a1, hw, dinv, b, w)


def _fin_body(a0_ref, a1_ref, hw_ref, dinv_ref, b_ref, wp1_ref, bp1_ref,
              wp2_ref, bp2_ref, out_ref):
    dinv = dinv_ref[...]
    h = jnp.maximum(
        dinv * (a0_ref[...] + a1_ref[...]) + dinv * dinv * hw_ref[...] + b_ref[...],
        0.0)
    t = jnp.tanh(jnp.dot(h, wp1_ref[...], preferred_element_type=jnp.float32)
                 + bp1_ref[...])
    out_ref[...] = jnp.tanh(
        jnp.dot(t, wp2_ref[...], preferred_element_type=jnp.float32)
        + bp2_ref[...])


def _fin(a0, a1, hw, dinv, b, Wp1, bp1, Wp2, bp2):
    return pl.pallas_call(
        _fin_body,
        out_shape=jax.ShapeDtypeStruct((N, 1), jnp.float32),
    )(a0, a1, hw, dinv, b, Wp1, bp1, Wp2, bp2)


def kernel(x, edge_index, S, R, We1, be1, We2, be2, Wg1, bg1, Wg2, bg2,
           Wp1, bp1, Wp2, bp2):
    src2 = edge_index[0].reshape(NW, E_PER_W)
    dst2 = edge_index[1].reshape(NW, E_PER_W)
    pad = E_PER_W_PAD - E_PER_W
    pad_dst = jnp.broadcast_to(N + jnp.arange(pad, dtype=jnp.int32),
                               (NW, pad))
    src3 = jnp.pad(src2, ((0, 0), (0, pad))).reshape(NW, NCH, CH)
    dst3 = jnp.concatenate([dst2, pad_dst], axis=1).reshape(NW, NCH, CH)
    zeros16 = jnp.zeros((N_PAD, 16), jnp.float32)
    zeros32 = jnp.zeros((N_PAD, 32), jnp.float32)
    ones16 = jnp.ones((CH, 16), jnp.float32)

    degacc = _deg_call_fn()(dst3, ones16, zeros16)

    A1 = We1[:NF]
    A2 = We1[NF:]
    hw1 = _tc1(x, S, R, A1, A2, be1.reshape(1, 64), We2,
               be2.reshape(1, 32), Wg1)

    g1, dinv = _prep(degacc[0, :N], degacc[1, :N], hw1)

    acc1 = _agg_call_fn()(src3, dst3, g1, zeros32)
    hw2, g2 = _mid(acc1[0, :N], acc1[1, :N], hw1, dinv, bg1.reshape(1, 32), Wg2)

    acc2 = _agg_call_fn()(src3, dst3, g2, zeros32)
    return _fin(acc2[0, :N], acc2[1, :N], hw2, dinv, bg2.reshape(1, 32),
                Wp1, bp1.reshape(1, 32), Wp2, bp2.reshape(1, 1))
